# baseline (speedup 1.0000x reference)
import functools

import jax
import jax.numpy as jnp
from jax import lax
from jax.experimental import pallas as pl
from jax.experimental.pallas import tpu as pltpu
from jax.experimental.pallas import tpu_sc as plsc

D = 768
E = 8
NSEG = D // 16
NC = 2
NS = 16
NW = NC * NS
CH = 32
B = 4
SC_TOKENS = 8192
NBUF = 4
BLK = 1024


def _sc_body(x_hbm, w_hbm, o_hbm, xbuf, wbuf, obuf, abuf, sems):
    c = lax.axis_index("c")
    s = lax.axis_index("s")
    wid = s * NC + c
    tk = SC_TOKENS // NW
    t_start = x_hbm.shape[0] - SC_TOKENS
    base = t_start + wid * tk
    obase = wid * tk
    nch = tk // CH
    iota = lax.iota(jnp.int32, 16)

    def in_copy(ci, p):
        return pltpu.make_async_copy(
            x_hbm.at[pl.ds(base + ci * CH, CH)], xbuf.at[p], sems.at[p])

    def out_copy(ci, p):
        dst = o_hbm.at[pl.ds(pl.multiple_of((obase + ci * CH) * E, 8), CH * E)]
        return pltpu.make_async_copy(obuf.at[p], dst, sems.at[2 + p])

    pltpu.sync_copy(w_hbm, wbuf)
    in_copy(0, 0).start()

    def chunk_body(ci, _):
        p = lax.rem(ci, 2)

        @pl.when(ci + 1 < nch)
        def _():
            in_copy(ci + 1, 1 - p).start()

        in_copy(ci, p).wait()

        @pl.when(ci >= 2)
        def _():
            out_copy(ci - 2, p).wait()

        @plsc.parallel_loop(0, CH // B)
        def tb_body(bi):
            t0 = bi * B
            accs = [[jnp.zeros((16,), jnp.float32) for _ in range(E)]
                    for _ in range(B)]
            for j in range(NSEG):
                wv = [wbuf[e, pl.ds(j * 16, 16)] for e in range(E)]
                for t in range(B):
                    xv = xbuf[p, t0 + t, pl.ds(j * 16, 16)]
                    for e in range(E):
                        accs[t][e] = accs[t][e] + xv * wv[e]
            for t in range(0, B, 2):
                off = (t0 + t) * E
                for dt in (0, 1):
                    for e in range(E):
                        abuf[off + 8 * dt + e] = accs[t + dt][e]
                cols = [plsc.load_gather(
                            abuf, [off + iota, jnp.full((16,), k, jnp.int32)])
                        for k in range(16)]
                while len(cols) > 1:
                    cols = [cols[i] + cols[i + 1]
                            for i in range(0, len(cols), 2)]
                obuf[p, pl.ds(pl.multiple_of(off, 16), 16)] = cols[0]

        out_copy(ci, p).start()
        return 0

    lax.fori_loop(0, nch, chunk_body, 0)
    out_copy(nch - 2, 0).wait()
    out_copy(nch - 1, 1).wait()


def _sc_call(x, W):
    mesh = plsc.VectorSubcoreMesh(core_axis_name="c", subcore_axis_name="s")
    k = functools.partial(
        pl.kernel,
        out_type=jax.ShapeDtypeStruct((SC_TOKENS * E,), jnp.float32),
        mesh=mesh,
        compiler_params=pltpu.CompilerParams(needs_layout_passes=False),
        scratch_types=[
            pltpu.VMEM((2, CH, D), jnp.float32),
            pltpu.VMEM((E, D), jnp.float32),
            pltpu.VMEM((2, CH * E), jnp.float32),
            pltpu.VMEM((CH * E, 16), jnp.float32),
            pltpu.SemaphoreType.DMA((4,)),
        ],
    )(_sc_body)
    return k(x, W).reshape(SC_TOKENS, E)


def _tc_body(x_hbm, wt_ref, o_ref, bufs, sems):
    nblk = o_ref.shape[0] // BLK
    wt = wt_ref[...]

    def start(i, p):
        pltpu.make_async_copy(
            x_hbm.at[pl.ds(i * BLK, BLK)], bufs.at[p], sems.at[p]).start()

    for b in range(NBUF):
        start(b, b)

    def step(i, _):
        p = lax.rem(i, NBUF)
        pltpu.make_async_copy(
            x_hbm.at[pl.ds(i * BLK, BLK)], bufs.at[p], sems.at[p]).wait()
        o_ref[pl.ds(i * BLK, BLK)] = jnp.dot(
            bufs[p], wt, preferred_element_type=jnp.float32)

        @pl.when(i + NBUF < nblk)
        def _():
            start(i + NBUF, p)

        return 0

    lax.fori_loop(0, nblk, step, 0)


def _tc_call(x, W):
    T = x.shape[0]
    Wt = W.T
    return pl.pallas_call(
        _tc_body,
        in_specs=[
            pl.BlockSpec(memory_space=pl.ANY),
            pl.BlockSpec(memory_space=pltpu.VMEM),
        ],
        out_specs=pl.BlockSpec(memory_space=pltpu.VMEM),
        out_shape=jax.ShapeDtypeStruct((T - SC_TOKENS, E), jnp.float32),
        scratch_shapes=[
            pltpu.VMEM((NBUF, BLK, D), jnp.float32),
            pltpu.SemaphoreType.DMA((NBUF,)),
        ],
    )(x, Wt)


def kernel(x, W):
    sc_out = _sc_call(x, W)
    tc_out = _tc_call(x, W)
    return jnp.concatenate([tc_out, sc_out], axis=0)


# speedup vs baseline: 1.3000x; 1.3000x over previous
import functools

import jax
import jax.numpy as jnp
from jax import lax
from jax.experimental import pallas as pl
from jax.experimental.pallas import tpu as pltpu
from jax.experimental.pallas import tpu_sc as plsc

D = 768
E = 8
NSEG = D // 16
NC = 2
NS = 16
NW = NC * NS
CH = 32
B = 4
SC_TOKENS = 2048
NBUF = 4
BLK = 1024


def _sc_body(x_hbm, w_hbm, o_hbm, xbuf, wbuf, obuf, abuf, sems):
    c = lax.axis_index("c")
    s = lax.axis_index("s")
    wid = s * NC + c
    tk = SC_TOKENS // NW
    t_start = x_hbm.shape[0] - SC_TOKENS
    base = t_start + wid * tk
    obase = wid * tk
    nch = tk // CH
    iota = lax.iota(jnp.int32, 16)

    def in_copy(ci, p):
        return pltpu.make_async_copy(
            x_hbm.at[pl.ds(base + ci * CH, CH)], xbuf.at[p], sems.at[p])

    def out_copy(ci, p):
        dst = o_hbm.at[pl.ds(pl.multiple_of((obase + ci * CH) * E, 8), CH * E)]
        return pltpu.make_async_copy(obuf.at[p], dst, sems.at[2 + p])

    pltpu.sync_copy(w_hbm, wbuf)
    in_copy(0, 0).start()

    def chunk_body(ci, _):
        p = lax.rem(ci, 2)

        @pl.when(ci + 1 < nch)
        def _():
            in_copy(ci + 1, 1 - p).start()

        in_copy(ci, p).wait()

        @pl.when(ci >= 2)
        def _():
            out_copy(ci - 2, p).wait()

        @plsc.parallel_loop(0, CH // B)
        def tb_body(bi):
            t0 = bi * B
            accs = [[jnp.zeros((16,), jnp.float32) for _ in range(E)]
                    for _ in range(B)]
            for j in range(NSEG):
                wv = [wbuf[e, pl.ds(j * 16, 16)] for e in range(E)]
                for t in range(B):
                    xv = xbuf[p, t0 + t, pl.ds(j * 16, 16)]
                    for e in range(E):
                        accs[t][e] = accs[t][e] + xv * wv[e]
            for t in range(0, B, 2):
                off = (t0 + t) * E
                for dt in (0, 1):
                    for e in range(E):
                        abuf[off + 8 * dt + e] = accs[t + dt][e]
                cols = [plsc.load_gather(
                            abuf, [off + iota, jnp.full((16,), k, jnp.int32)])
                        for k in range(16)]
                while len(cols) > 1:
                    cols = [cols[i] + cols[i + 1]
                            for i in range(0, len(cols), 2)]
                obuf[p, pl.ds(pl.multiple_of(off, 16), 16)] = cols[0]

        out_copy(ci, p).start()
        return 0

    lax.fori_loop(0, nch, chunk_body, 0)
    out_copy(nch - 2, 0).wait()
    out_copy(nch - 1, 1).wait()


def _sc_call(x, W):
    mesh = plsc.VectorSubcoreMesh(core_axis_name="c", subcore_axis_name="s")
    k = functools.partial(
        pl.kernel,
        out_type=jax.ShapeDtypeStruct((SC_TOKENS * E,), jnp.float32),
        mesh=mesh,
        compiler_params=pltpu.CompilerParams(needs_layout_passes=False),
        scratch_types=[
            pltpu.VMEM((2, CH, D), jnp.float32),
            pltpu.VMEM((E, D), jnp.float32),
            pltpu.VMEM((2, CH * E), jnp.float32),
            pltpu.VMEM((CH * E, 16), jnp.float32),
            pltpu.SemaphoreType.DMA((4,)),
        ],
    )(_sc_body)
    return k(x, W).reshape(SC_TOKENS, E)


def _tc_body(x_hbm, wt_ref, o_ref, bufs, sems):
    nblk = o_ref.shape[0] // BLK
    wt = wt_ref[...]

    def start(i, p):
        pltpu.make_async_copy(
            x_hbm.at[pl.ds(i * BLK, BLK)], bufs.at[p], sems.at[p]).start()

    for b in range(NBUF):
        start(b, b)

    def step(i, _):
        p = lax.rem(i, NBUF)
        pltpu.make_async_copy(
            x_hbm.at[pl.ds(i * BLK, BLK)], bufs.at[p], sems.at[p]).wait()
        o_ref[pl.ds(i * BLK, BLK)] = jnp.dot(
            bufs[p], wt, preferred_element_type=jnp.float32)

        @pl.when(i + NBUF < nblk)
        def _():
            start(i + NBUF, p)

        return 0

    lax.fori_loop(0, nblk, step, 0)


def _tc_call(x, W):
    T = x.shape[0]
    Wt = W.T
    return pl.pallas_call(
        _tc_body,
        in_specs=[
            pl.BlockSpec(memory_space=pl.ANY),
            pl.BlockSpec(memory_space=pltpu.VMEM),
        ],
        out_specs=pl.BlockSpec(memory_space=pltpu.VMEM),
        out_shape=jax.ShapeDtypeStruct((T - SC_TOKENS, E), jnp.float32),
        scratch_shapes=[
            pltpu.VMEM((NBUF, BLK, D), jnp.float32),
            pltpu.SemaphoreType.DMA((NBUF,)),
        ],
    )(x, Wt)


def kernel(x, W):
    sc_out = _sc_call(x, W)
    tc_out = _tc_call(x, W)
    return jnp.concatenate([tc_out, sc_out], axis=0)
